# baseline (speedup 1.0000x reference)
import functools

import jax
import jax.numpy as jnp
from jax import lax
from jax.experimental import pallas as pl
from jax.experimental.pallas import tpu as pltpu
from jax.experimental.pallas import tpu_sc as plsc

MAX_REL = 100
EMB = 64
IDS_MINOR = 128


@functools.lru_cache(maxsize=None)
def _build(n_ids_rows: int, n_table_rows: int):
    info = plsc.get_sparse_core_info()
    L = info.num_lanes
    num_workers = info.num_cores * info.num_subcores
    rows_per_worker = n_ids_rows // num_workers
    n_blocks = IDS_MINOR // L

    mesh = plsc.VectorSubcoreMesh(core_axis_name="c", subcore_axis_name="s")

    @functools.partial(
        pl.kernel,
        mesh=mesh,
        out_type=jax.ShapeDtypeStruct((n_ids_rows * IDS_MINOR, EMB), jnp.float32),
        scratch_types=[
            pltpu.VMEM((rows_per_worker, IDS_MINOR), jnp.int32),
            pltpu.VMEM((n_table_rows, EMB), jnp.float32),
            pltpu.VMEM((IDS_MINOR, EMB), jnp.float32),
            pltpu.VMEM((IDS_MINOR, EMB), jnp.float32),
            pltpu.SemaphoreType.DMA,
            pltpu.SemaphoreType.DMA,
        ],
        compiler_params=pltpu.CompilerParams(
            use_tc_tiling_on_sc=False, needs_layout_passes=False
        ),
    )
    def k(ids_hbm, w_hbm, out_hbm, idx_v, table_v, rows0, rows1, osem0, osem1):
        wid = lax.axis_index("s") * info.num_cores + lax.axis_index("c")
        row0 = wid * rows_per_worker
        rows_bufs = (rows0, rows1)
        osems = (osem0, osem1)

        pltpu.sync_copy(w_hbm, table_v)
        pltpu.sync_copy(ids_hbm.at[pl.ds(row0, rows_per_worker)], idx_v)

        def clamp_row(r, carry):
            for kk in range(IDS_MINOR // L):
                sl = pl.ds(kk * L, L)
                idx_v[r, sl] = jnp.minimum(idx_v[r, sl], MAX_REL)
            return carry

        lax.fori_loop(0, rows_per_worker, clamp_row, 0)

        dst_rows = [
            jax.lax.iota(jnp.int32, L) + b * L for b in range(n_blocks)
        ]

        def assemble_chunk(ch, buf):
            ivecs = [idx_v[ch, pl.ds(b * L, L)] for b in range(n_blocks)]
            for c in range(EMB):
                csplat = jnp.full((L,), c, jnp.int32)
                for b in range(n_blocks):
                    g = plsc.load_gather(table_v, [ivecs[b], csplat])
                    plsc.store_scatter(buf, [dst_rows[b], csplat], g)

        def writeback(ch, buf, sem):
            return pltpu.make_async_copy(
                buf,
                out_hbm.at[pl.ds((row0 + ch) * IDS_MINOR, IDS_MINOR)],
                sem,
            )

        for b in (0, 1):
            assemble_chunk(b, rows_bufs[b])
            writeback(b, rows_bufs[b], osems[b]).start()

        def body(g, carry):
            for b in (0, 1):
                ch = 2 * g + b
                writeback(ch - 2, rows_bufs[b], osems[b]).wait()
                assemble_chunk(ch, rows_bufs[b])
                writeback(ch, rows_bufs[b], osems[b]).start()
            return carry

        lax.fori_loop(1, rows_per_worker // 2, body, 0)

        for b in (0, 1):
            writeback(rows_per_worker - 2 + b, rows_bufs[b], osems[b]).wait()

    return k


def kernel(relative_position_ids, weight):
    b, h = relative_position_ids.shape
    ids2 = relative_position_ids.astype(jnp.int32).reshape(-1, IDS_MINOR)
    out = _build(ids2.shape[0], weight.shape[0])(ids2, weight)
    return out.reshape(b, h, EMB)


# speedup vs baseline: 1.0095x; 1.0095x over previous
import functools

import jax
import jax.numpy as jnp
from jax import lax
from jax.experimental import pallas as pl
from jax.experimental.pallas import tpu as pltpu
from jax.experimental.pallas import tpu_sc as plsc

MAX_REL = 100
EMB = 64
IDS_MINOR = 128


@functools.lru_cache(maxsize=None)
def _build(n_ids_rows: int, n_table_rows: int):
    info = plsc.get_sparse_core_info()
    L = info.num_lanes
    num_workers = info.num_cores * info.num_subcores
    rows_per_worker = n_ids_rows // num_workers
    n_blocks = IDS_MINOR // L

    mesh = plsc.VectorSubcoreMesh(core_axis_name="c", subcore_axis_name="s")

    @functools.partial(
        pl.kernel,
        mesh=mesh,
        out_type=jax.ShapeDtypeStruct((n_ids_rows * IDS_MINOR, EMB), jnp.float32),
        scratch_types=[
            pltpu.VMEM((rows_per_worker, IDS_MINOR), jnp.int32),
            pltpu.VMEM((n_table_rows, EMB), jnp.float32),
            pltpu.VMEM((IDS_MINOR, EMB), jnp.float32),
            pltpu.VMEM((IDS_MINOR, EMB), jnp.float32),
            pltpu.SemaphoreType.DMA,
            pltpu.SemaphoreType.DMA,
        ],
        compiler_params=pltpu.CompilerParams(
            use_tc_tiling_on_sc=False, needs_layout_passes=False
        ),
    )
    def k(ids_hbm, w_hbm, out_hbm, idx_v, table_v, rows0, rows1, osem0, osem1):
        wid = lax.axis_index("s") * info.num_cores + lax.axis_index("c")
        row0 = wid * rows_per_worker
        rows_bufs = (rows0, rows1)
        osems = (osem0, osem1)

        pltpu.sync_copy(w_hbm, table_v)
        pltpu.sync_copy(ids_hbm.at[pl.ds(row0, rows_per_worker)], idx_v)

        def clamp_row(r, carry):
            for kk in range(IDS_MINOR // L):
                sl = pl.ds(kk * L, L)
                idx_v[r, sl] = jnp.minimum(idx_v[r, sl], MAX_REL)
            return carry

        lax.fori_loop(0, rows_per_worker, clamp_row, 0)

        dst_rows = [
            jax.lax.iota(jnp.int32, L) + b * L for b in range(n_blocks)
        ]

        def assemble_chunk(ch, buf):
            ivecs = [idx_v[ch, pl.ds(b * L, L)] for b in range(n_blocks)]
            for c in range(EMB):
                csplat = jnp.full((L,), c, jnp.int32)
                gs = [
                    plsc.load_gather(table_v, [ivecs[b], csplat])
                    for b in range(n_blocks)
                ]
                for b in range(n_blocks):
                    plsc.store_scatter(buf, [dst_rows[b], csplat], gs[b])

        def writeback(ch, buf, sem):
            return pltpu.make_async_copy(
                buf,
                out_hbm.at[pl.ds((row0 + ch) * IDS_MINOR, IDS_MINOR)],
                sem,
            )

        for b in (0, 1):
            assemble_chunk(b, rows_bufs[b])
            writeback(b, rows_bufs[b], osems[b]).start()

        def body(g, carry):
            for b in (0, 1):
                ch = 2 * g + b
                writeback(ch - 2, rows_bufs[b], osems[b]).wait()
                assemble_chunk(ch, rows_bufs[b])
                writeback(ch, rows_bufs[b], osems[b]).start()
            return carry

        lax.fori_loop(1, rows_per_worker // 2, body, 0)

        for b in (0, 1):
            writeback(rows_per_worker - 2 + b, rows_bufs[b], osems[b]).wait()

    return k


def kernel(relative_position_ids, weight):
    b, h = relative_position_ids.shape
    ids2 = relative_position_ids.astype(jnp.int32).reshape(-1, IDS_MINOR)
    out = _build(ids2.shape[0], weight.shape[0])(ids2, weight)
    return out.reshape(b, h, EMB)


# speedup vs baseline: 3.3097x; 3.2786x over previous
import functools

import jax
import jax.numpy as jnp
from jax import lax
from jax.experimental import pallas as pl
from jax.experimental.pallas import tpu as pltpu
from jax.experimental.pallas import tpu_sc as plsc

MAX_REL = 100
EMB = 64
PADDED = 72
IDS_MINOR = 128


@functools.lru_cache(maxsize=None)
def _build(n_ids_rows: int, n_table_rows: int):
    info = plsc.get_sparse_core_info()
    num_workers = info.num_cores * info.num_subcores
    rows_per_worker = n_ids_rows // num_workers
    rows_per_chunk = 4
    n_chunks = rows_per_worker // rows_per_chunk
    chunk = rows_per_chunk * IDS_MINOR

    mesh = plsc.VectorSubcoreMesh(core_axis_name="c", subcore_axis_name="s")

    @functools.partial(
        pl.kernel,
        mesh=mesh,
        out_type=jax.ShapeDtypeStruct((n_ids_rows * IDS_MINOR, EMB), jnp.float32),
        scratch_types=[
            pltpu.VMEM((rows_per_worker, IDS_MINOR), jnp.int32),
            pltpu.VMEM((chunk, PADDED), jnp.float32),
            pltpu.VMEM((chunk, PADDED), jnp.float32),
            pltpu.VMEM_SHARED((n_table_rows, PADDED), jnp.float32),
            pltpu.SemaphoreType.DMA,
            pltpu.SemaphoreType.DMA,
            pltpu.SemaphoreType.DMA,
        ],
        compiler_params=pltpu.CompilerParams(use_tc_tiling_on_sc=False),
    )
    def k(ids_hbm, w_hbm, out_hbm, idx_v, rows0, rows1, table_sh, gsem, osem0, osem1):
        sid = lax.axis_index("s")
        wid = sid * info.num_cores + lax.axis_index("c")
        row0 = wid * rows_per_worker
        out0 = row0 * IDS_MINOR
        rows_bufs = (rows0, rows1)
        osems = (osem0, osem1)

        @pl.when(sid == 0)
        def _():
            pltpu.sync_copy(w_hbm, table_sh)

        pltpu.sync_copy(ids_hbm.at[pl.ds(row0, rows_per_worker)], idx_v)

        def clamp_row(r, carry):
            for kk in range(IDS_MINOR // 16):
                sl = pl.ds(kk * 16, 16)
                idx_v[r, sl] = jnp.minimum(idx_v[r, sl], MAX_REL)
            return carry

        lax.fori_loop(0, rows_per_worker, clamp_row, 0)
        plsc.subcore_barrier()

        def gather_chunk(ch, buf):
            copies = [
                pltpu.async_copy(
                    table_sh.at[idx_v.at[ch * rows_per_chunk + j]],
                    buf.at[pl.ds(j * IDS_MINOR, IDS_MINOR)],
                    gsem,
                )
                for j in range(rows_per_chunk)
            ]
            for c in copies:
                c.wait()

        def writeback(ch, buf, sem):
            return pltpu.make_async_copy(
                buf.at[:, pl.ds(0, EMB)],
                out_hbm.at[pl.ds(out0 + ch * chunk, chunk)],
                sem,
            )

        for b in (0, 1):
            gather_chunk(b, rows_bufs[b])
            writeback(b, rows_bufs[b], osems[b]).start()

        def body(g, carry):
            for b in (0, 1):
                ch = 2 * g + b
                writeback(ch - 2, rows_bufs[b], osems[b]).wait()
                gather_chunk(ch, rows_bufs[b])
                writeback(ch, rows_bufs[b], osems[b]).start()
            return carry

        lax.fori_loop(1, n_chunks // 2, body, 0)

        for b in (0, 1):
            writeback(n_chunks - 2 + b, rows_bufs[b], osems[b]).wait()

    return k


def kernel(relative_position_ids, weight):
    b, h = relative_position_ids.shape
    ids2 = relative_position_ids.astype(jnp.int32).reshape(-1, IDS_MINOR)
    w_padded = jnp.pad(weight, ((0, 0), (0, PADDED - EMB)))
    out = _build(ids2.shape[0], weight.shape[0])(ids2, w_padded)
    return out.reshape(b, h, EMB)
